# baseline (speedup 1.0000x reference)
import functools

import jax
import jax.numpy as jnp
from jax import lax
from jax.experimental import pallas as pl
from jax.experimental.pallas import tpu as pltpu
from jax.experimental.pallas import tpu_sc as plsc

EMB = 64
IMG = 28
BATCH = 16384

_info = plsc.get_sparse_core_info()
_NC = _info.num_cores
_NS = _info.num_subcores
_NW = _NC * _NS
_BPW = BATCH // _NW
_CH = 128
_NCH = _BPW // _CH

_mesh = plsc.VectorSubcoreMesh(core_axis_name="c", subcore_axis_name="s")


@functools.partial(
    pl.kernel,
    mesh=_mesh,
    out_type=jax.ShapeDtypeStruct((BATCH, EMB), jnp.bfloat16),
    scratch_types=[
        pltpu.VMEM((_NCH, _CH), jnp.int32),
        pltpu.VMEM((_BPW, EMB), jnp.bfloat16),
        pltpu.SemaphoreType.DMA,
    ],
    compiler_params=pltpu.CompilerParams(use_tc_tiling_on_sc=False),
)
def _sc_gather(idx_hbm, table_hbm, out_hbm, idx_v, rows_v, sem):
    wid = lax.axis_index("s") * _NC + lax.axis_index("c")
    pltpu.sync_copy(idx_hbm.at[wid], idx_v)
    copies = [
        pltpu.async_copy(
            table_hbm.at[idx_v.at[j]],
            rows_v.at[pl.ds(j * _CH, _CH)],
            sem,
        )
        for j in range(_NCH)
    ]
    for c in copies:
        c.wait()
    pltpu.sync_copy(rows_v, out_hbm.at[pl.ds(wid * _BPW, _BPW)])


_BN = 2048


def _mm_body(w_ref, emb_ref, b_ref, out_ref):
    out_ref[...] = (
        lax.dot_general(
            w_ref[...],
            emb_ref[...],
            (((0,), (1,)), ((), ())),
            preferred_element_type=jnp.float32,
        )
        + b_ref[...]
    )


def kernel(x, table, W, b):
    idx = x.astype(jnp.int32).reshape(_NW, _NCH, _CH)
    emb = _sc_gather(idx, table.astype(jnp.bfloat16))
    outT = pl.pallas_call(
        _mm_body,
        grid=(BATCH // _BN,),
        in_specs=[
            pl.BlockSpec((EMB, IMG * IMG), lambda i: (0, 0)),
            pl.BlockSpec((_BN, EMB), lambda i: (i, 0)),
            pl.BlockSpec((IMG * IMG, 1), lambda i: (0, 0)),
        ],
        out_specs=pl.BlockSpec((IMG * IMG, _BN), lambda i: (0, i)),
        out_shape=jax.ShapeDtypeStruct((IMG * IMG, BATCH), jnp.float32),
    )(W, emb, b.reshape(IMG * IMG, 1))
    return outT.T.reshape(BATCH, IMG, IMG)


# speedup vs baseline: 1.6153x; 1.6153x over previous
import functools

import jax
import jax.numpy as jnp
from jax import lax
from jax.experimental import pallas as pl
from jax.experimental.pallas import tpu as pltpu
from jax.experimental.pallas import tpu_sc as plsc

EMB = 64
IMG = 28
BATCH = 16384

_info = plsc.get_sparse_core_info()
_NC = _info.num_cores
_NS = _info.num_subcores
_NW = _NC * _NS
_BPW = BATCH // _NW
_CH = 64
_NCH = _BPW // _CH
_FB = 16

_mesh = plsc.VectorSubcoreMesh(core_axis_name="c", subcore_axis_name="s")


@functools.partial(
    pl.kernel,
    mesh=_mesh,
    out_type=jax.ShapeDtypeStruct((BATCH, 8, EMB), jnp.bfloat16),
    scratch_types=[
        pltpu.VMEM((_NCH, _CH), jnp.int32),
        pltpu.VMEM((_CH, 8, EMB), jnp.bfloat16),
        pltpu.VMEM((_CH, 8, EMB), jnp.bfloat16),
        pltpu.SemaphoreType.DMA,
        pltpu.SemaphoreType.DMA,
    ],
)
def _sc_gather(grp_hbm, table_hbm, out_hbm, idx_v, rows0, rows1, sem0, sem1):
    wid = lax.axis_index("s") * _NC + lax.axis_index("c")
    base = wid * _BPW
    pltpu.sync_copy(grp_hbm.at[wid], idx_v)
    bufs = (rows0, rows1)
    sems = (sem0, sem1)

    def gather_chunk(c, buf, sem):
        for g in range(_CH // _FB):
            vec = idx_v[c, pl.ds(g * _FB, _FB)]
            cps = []
            for k in range(_FB):
                i = pl.multiple_of(vec[k], 8)
                cps.append(
                    pltpu.async_copy(
                        table_hbm.at[pl.ds(i, 8)],
                        buf.at[g * _FB + k],
                        sem,
                    )
                )
            for cp in cps:
                cp.wait()

    for c in range(_NCH):
        buf = bufs[c % 2]
        gather_chunk(c, buf, sems[c % 2])
        pltpu.sync_copy(buf, out_hbm.at[pl.ds(base + c * _CH, _CH)])


_BN = 2048


def _mm_body(emb3_ref, subs_ref, w_ref, b_ref, out_ref):
    subs = subs_ref[...]
    emb = emb3_ref[:, 0, :].astype(jnp.float32) * (subs == 0).astype(jnp.float32)
    for s in range(1, 8):
        emb = emb + emb3_ref[:, s, :].astype(jnp.float32) * (subs == s).astype(
            jnp.float32
        )
    out_ref[...] = (
        lax.dot_general(
            w_ref[...],
            emb,
            (((0,), (1,)), ((), ())),
            preferred_element_type=jnp.float32,
        )
        + b_ref[...]
    )


def kernel(x, table, W, b):
    xi = x.astype(jnp.int32)
    grp = (xi & ~7).reshape(_NW, _NCH, _CH)
    subs = (xi & 7).reshape(BATCH, 1)
    emb3 = _sc_gather(grp, table.astype(jnp.bfloat16))
    outT = pl.pallas_call(
        _mm_body,
        grid=(BATCH // _BN,),
        in_specs=[
            pl.BlockSpec((_BN, 8, EMB), lambda i: (i, 0, 0)),
            pl.BlockSpec((_BN, 1), lambda i: (i, 0)),
            pl.BlockSpec((EMB, IMG * IMG), lambda i: (0, 0)),
            pl.BlockSpec((IMG * IMG, 1), lambda i: (0, 0)),
        ],
        out_specs=pl.BlockSpec((IMG * IMG, _BN), lambda i: (0, i)),
        out_shape=jax.ShapeDtypeStruct((IMG * IMG, BATCH), jnp.float32),
    )(emb3, subs, W, b.reshape(IMG * IMG, 1))
    return outT.T.reshape(BATCH, IMG, IMG)
